# Initial kernel scaffold; baseline (speedup 1.0000x reference)
import jax
import jax.numpy as jnp
from jax.experimental import pallas as pl


def kernel(x, edge_index, W1, b1, W2, b2, W3, b3):
    raise NotImplementedError("write your pallas kernel here")


# speedup vs baseline: 7.7295x; 7.7295x over previous
import functools

import jax
import jax.numpy as jnp
from jax import lax
from jax.experimental import pallas as pl
from jax.experimental.pallas import tpu as pltpu
from jax.experimental.pallas import tpu_sc as plsc

N = 10000
D = 128
E = 320000
NC = 2
NS = 16
K = 80
EDGES_PER_W = E // (NC * NS)
NCH = EDGES_PER_W // K
NPAIR = (NCH - 1) // 2
ROWS_PER_S = 624
ROWS_TAIL = N - NS * ROWS_PER_S


def _build_sc_aggregate():
    mesh = plsc.VectorSubcoreMesh(core_axis_name="c", subcore_axis_name="s")

    @functools.partial(
        pl.kernel,
        out_type=jax.ShapeDtypeStruct((NC, N, D), jnp.float32),
        mesh=mesh,
        scratch_types=[
            pltpu.VMEM((2, K), jnp.int32),
            pltpu.VMEM((2, K), jnp.int32),
            pltpu.VMEM((2, K, D), jnp.float32),
            pltpu.VMEM_SHARED((N, D), jnp.float32),
            pltpu.SemaphoreType.DMA,
            pltpu.SemaphoreType.DMA,
        ],
    )
    def agg(h_hbm, edge_hbm, out_hbm, src_v, dst_v, rows_v, acc, sem0, sem1):
        c = lax.axis_index("c")
        s = lax.axis_index("s")
        wid = s * NC + c
        base = wid * EDGES_PER_W
        sems = (sem0, sem1)

        r0 = s * ROWS_PER_S
        pltpu.sync_copy(h_hbm.at[pl.ds(r0, ROWS_PER_S)],
                        acc.at[pl.ds(r0, ROWS_PER_S)])

        @pl.when(s == NS - 1)
        def _():
            pltpu.sync_copy(h_hbm.at[pl.ds(NS * ROWS_PER_S, ROWS_TAIL)],
                            acc.at[pl.ds(NS * ROWS_PER_S, ROWS_TAIL)])

        plsc.subcore_barrier()

        def load_and_fire(ci, b):
            off = base + ci * K
            pltpu.sync_copy(edge_hbm.at[pl.ds(off, K)], src_v.at[b])
            pltpu.sync_copy(edge_hbm.at[pl.ds(E + off, K)], dst_v.at[b])
            pltpu.make_async_copy(h_hbm.at[src_v.at[b]], rows_v.at[b],
                                  sems[b]).start()

        def drain_and_add(b):
            pltpu.make_async_copy(h_hbm.at[src_v.at[b]], rows_v.at[b],
                                  sems[b]).wait()
            pltpu.sync_copy(rows_v.at[b], acc.at[dst_v.at[b]], add=True)

        load_and_fire(0, 0)

        @pl.loop(0, NPAIR)
        def _(p):
            load_and_fire(2 * p + 1, 1)
            drain_and_add(0)
            load_and_fire(2 * p + 2, 0)
            drain_and_add(1)

        drain_and_add(0)
        plsc.subcore_barrier()

        pltpu.sync_copy(acc.at[pl.ds(r0, ROWS_PER_S)],
                        out_hbm.at[c, pl.ds(r0, ROWS_PER_S)])

        @pl.when(s == NS - 1)
        def _():
            pltpu.sync_copy(acc.at[pl.ds(NS * ROWS_PER_S, ROWS_TAIL)],
                            out_hbm.at[c, pl.ds(NS * ROWS_PER_S, ROWS_TAIL)])

    return agg


_sc_aggregate = _build_sc_aggregate()

_CDIMS = (((1,), (1,)), ((), ()))
BLK = 2000


def _mlp1_body(p_ref, x_ref, w1_ref, b1_ref, w2_ref, b2_ref, o_ref):
    agg = p_ref[0] + p_ref[1] - x_ref[...]
    t = lax.dot_general(agg, w1_ref[...], _CDIMS,
                        preferred_element_type=jnp.float32) + b1_ref[...]
    t = jnp.maximum(t, 0.0)
    h = lax.dot_general(t, w2_ref[...], _CDIMS,
                        preferred_element_type=jnp.float32) + b2_ref[...]
    o_ref[...] = jnp.maximum(h, 0.0)


def _mlp2_body(q_ref, h_ref, w3_ref, b3_ref, o_ref):
    agg = q_ref[0] + q_ref[1] - h_ref[...]
    t = lax.dot_general(agg, w3_ref[...], _CDIMS,
                        preferred_element_type=jnp.float32) + b3_ref[...]
    t = jnp.maximum(t, 0.0)
    m = jnp.max(t, axis=1, keepdims=True)
    lse = m + jnp.log(jnp.sum(jnp.exp(t - m), axis=1, keepdims=True))
    o_ref[...] = t - lse


def _mlp1(p, x, w1, b1, w2, b2):
    return pl.pallas_call(
        _mlp1_body,
        grid=(N // BLK,),
        in_specs=[
            pl.BlockSpec((NC, BLK, D), lambda i: (0, i, 0)),
            pl.BlockSpec((BLK, D), lambda i: (i, 0)),
            pl.BlockSpec((D, D), lambda i: (0, 0)),
            pl.BlockSpec((1, D), lambda i: (0, 0)),
            pl.BlockSpec((D, D), lambda i: (0, 0)),
            pl.BlockSpec((1, D), lambda i: (0, 0)),
        ],
        out_specs=pl.BlockSpec((BLK, D), lambda i: (i, 0)),
        out_shape=jax.ShapeDtypeStruct((N, D), jnp.float32),
    )(p, x, w1, b1.reshape(1, D), w2, b2.reshape(1, D))


def _mlp2(q, h, w3, b3):
    return pl.pallas_call(
        _mlp2_body,
        grid=(N // BLK,),
        in_specs=[
            pl.BlockSpec((NC, BLK, D), lambda i: (0, i, 0)),
            pl.BlockSpec((BLK, D), lambda i: (i, 0)),
            pl.BlockSpec((D, D), lambda i: (0, 0)),
            pl.BlockSpec((1, D), lambda i: (0, 0)),
        ],
        out_specs=pl.BlockSpec((BLK, D), lambda i: (i, 0)),
        out_shape=jax.ShapeDtypeStruct((N, D), jnp.float32),
    )(q, h, w3, b3.reshape(1, D))


def kernel(x, edge_index, W1, b1, W2, b2, W3, b3):
    edges_flat = edge_index.reshape(2 * E)
    p = _sc_aggregate(x, edges_flat)
    h1 = _mlp1(p, x, W1, b1, W2, b2)
    q = _sc_aggregate(h1, edges_flat)
    return _mlp2(q, h1, W3, b3)


# speedup vs baseline: 10.7801x; 1.3947x over previous
import functools

import jax
import jax.numpy as jnp
from jax import lax
from jax.experimental import pallas as pl
from jax.experimental.pallas import tpu as pltpu
from jax.experimental.pallas import tpu_sc as plsc

N = 10000
D = 128
E = 320000
NC = 2
NS = 16
K = 80
EDGES_PER_W = E // (NC * NS)
NCH = EDGES_PER_W // K
NW = NC * NS
NBUF = 2
ROWS_PER_S = 624
ROWS_TAIL = N - NS * ROWS_PER_S


def _build_sc_aggregate():
    mesh = plsc.VectorSubcoreMesh(core_axis_name="c", subcore_axis_name="s")

    @functools.partial(
        pl.kernel,
        out_type=jax.ShapeDtypeStruct((NC, N, D), jnp.float32),
        mesh=mesh,
        scratch_types=[
            pltpu.VMEM((EDGES_PER_W,), jnp.int32),
            pltpu.VMEM((NCH, K), jnp.int32),
            pltpu.VMEM((NBUF, K, D), jnp.float32),
            pltpu.VMEM_SHARED((N, D), jnp.float32),
            pltpu.SemaphoreType.DMA,
            pltpu.SemaphoreType.DMA,
        ],
    )
    def agg(h_hbm, src_hbm, dst_hbm, out_hbm, src_v, dst_v, rows_v, acc,
            sem0, sem1):
        c = lax.axis_index("c")
        s = lax.axis_index("s")
        wid = s * NC + c
        sems = (sem0, sem1)

        pltpu.sync_copy(src_hbm.at[wid], src_v)
        pltpu.sync_copy(dst_hbm.at[wid], dst_v)

        r0 = s * ROWS_PER_S
        pltpu.sync_copy(h_hbm.at[pl.ds(r0, ROWS_PER_S)],
                        acc.at[pl.ds(r0, ROWS_PER_S)])

        @pl.when(s == NS - 1)
        def _():
            pltpu.sync_copy(h_hbm.at[pl.ds(NS * ROWS_PER_S, ROWS_TAIL)],
                            acc.at[pl.ds(NS * ROWS_PER_S, ROWS_TAIL)])

        plsc.subcore_barrier()

        def fire(ci, b):
            pltpu.make_async_copy(h_hbm.at[src_v.at[pl.ds(ci * K, K)]],
                                  rows_v.at[b], sems[b]).start()

        def drain_and_add(ci, b):
            pltpu.make_async_copy(h_hbm.at[src_v.at[pl.ds(ci * K, K)]],
                                  rows_v.at[b], sems[b]).wait()
            pltpu.sync_copy(rows_v.at[b], acc.at[dst_v.at[ci]], add=True)

        for b in range(NBUF):
            fire(b, b)

        @pl.loop(0, (NCH - NBUF - 1) // NBUF)
        def _(p):
            for b in range(NBUF):
                ci = NBUF * p + b
                drain_and_add(ci, b)
                fire(ci + NBUF, b)

        base = NCH - NBUF - 1
        drain_and_add(base, 0)
        fire(NCH - 1, 0)
        for b in range(1, NBUF):
            drain_and_add(base + b, b)
        drain_and_add(NCH - 1, 0)
        plsc.subcore_barrier()

        pltpu.sync_copy(acc.at[pl.ds(r0, ROWS_PER_S)],
                        out_hbm.at[c, pl.ds(r0, ROWS_PER_S)])

        @pl.when(s == NS - 1)
        def _():
            pltpu.sync_copy(acc.at[pl.ds(NS * ROWS_PER_S, ROWS_TAIL)],
                            out_hbm.at[c, pl.ds(NS * ROWS_PER_S, ROWS_TAIL)])

    return agg


_sc_aggregate = _build_sc_aggregate()

_CDIMS = (((1,), (1,)), ((), ()))
BLK = 2000


def _mlp1_body(p_ref, x_ref, w1_ref, b1_ref, w2_ref, b2_ref, o_ref):
    agg = p_ref[0] + p_ref[1] - x_ref[...]
    t = lax.dot_general(agg, w1_ref[...], _CDIMS,
                        preferred_element_type=jnp.float32) + b1_ref[...]
    t = jnp.maximum(t, 0.0)
    h = lax.dot_general(t, w2_ref[...], _CDIMS,
                        preferred_element_type=jnp.float32) + b2_ref[...]
    o_ref[...] = jnp.maximum(h, 0.0)


def _mlp2_body(q_ref, h_ref, w3_ref, b3_ref, o_ref):
    agg = q_ref[0] + q_ref[1] - h_ref[...]
    t = lax.dot_general(agg, w3_ref[...], _CDIMS,
                        preferred_element_type=jnp.float32) + b3_ref[...]
    t = jnp.maximum(t, 0.0)
    m = jnp.max(t, axis=1, keepdims=True)
    lse = m + jnp.log(jnp.sum(jnp.exp(t - m), axis=1, keepdims=True))
    o_ref[...] = t - lse


def _mlp1(p, x, w1, b1, w2, b2):
    return pl.pallas_call(
        _mlp1_body,
        grid=(N // BLK,),
        in_specs=[
            pl.BlockSpec((NC, BLK, D), lambda i: (0, i, 0)),
            pl.BlockSpec((BLK, D), lambda i: (i, 0)),
            pl.BlockSpec((D, D), lambda i: (0, 0)),
            pl.BlockSpec((1, D), lambda i: (0, 0)),
            pl.BlockSpec((D, D), lambda i: (0, 0)),
            pl.BlockSpec((1, D), lambda i: (0, 0)),
        ],
        out_specs=pl.BlockSpec((BLK, D), lambda i: (i, 0)),
        out_shape=jax.ShapeDtypeStruct((N, D), jnp.float32),
    )(p, x, w1, b1.reshape(1, D), w2, b2.reshape(1, D))


def _mlp2(q, h, w3, b3):
    return pl.pallas_call(
        _mlp2_body,
        grid=(N // BLK,),
        in_specs=[
            pl.BlockSpec((NC, BLK, D), lambda i: (0, i, 0)),
            pl.BlockSpec((BLK, D), lambda i: (i, 0)),
            pl.BlockSpec((D, D), lambda i: (0, 0)),
            pl.BlockSpec((1, D), lambda i: (0, 0)),
        ],
        out_specs=pl.BlockSpec((BLK, D), lambda i: (i, 0)),
        out_shape=jax.ShapeDtypeStruct((N, D), jnp.float32),
    )(q, h, w3, b3.reshape(1, D))


def kernel(x, edge_index, W1, b1, W2, b2, W3, b3):
    src = edge_index[0].reshape(NW, EDGES_PER_W)
    dst = edge_index[1].reshape(NW, NCH, K)
    p = _sc_aggregate(x, src, dst)
    h1 = _mlp1(p, x, W1, b1, W2, b2)
    q = _sc_aggregate(h1, src, dst)
    return _mlp2(q, h1, W3, b3)
